# Initial kernel scaffold; baseline (speedup 1.0000x reference)
import jax
import jax.numpy as jnp
from jax.experimental import pallas as pl


def kernel(inputs, W):
    raise NotImplementedError("write your pallas kernel here")


# speedup vs baseline: 1.3967x; 1.3967x over previous
import functools

import jax
import jax.numpy as jnp
from jax import lax
from jax.experimental import pallas as pl
from jax.experimental.pallas import tpu as pltpu
from jax.experimental.pallas import tpu_sc as plsc

FIELDS = 26
DIM = 16
ROW = FIELDS * DIM
NC, NS = 2, 16
NW = NC * NS

G = 4
CROWS = G * FIELDS

PAIRS = [(i, j) for i in range(FIELDS - 1) for j in range(i + 1, FIELDS)]


def _lane_total(v):
    dnums = lax.GatherDimensionNumbers(
        offset_dims=(), collapsed_slice_dims=(0,), start_index_map=(0,))
    for sh in (8, 4, 2, 1):
        perm = jnp.bitwise_xor(lax.iota(jnp.int32, 16), sh)
        v = v + lax.gather(
            v, perm[:, None], dimension_numbers=dnums, slice_sizes=(1,),
            mode=lax.GatherScatterMode.PROMISE_IN_BOUNDS)
    return v


def _ffm_body(batch, tbl, idxs, out, idxbuf, ebuf, outbuf, sem0, sem1):
    epw = batch // NW
    chunks = epw // G
    wid = lax.axis_index("s") * NC + lax.axis_index("c")

    pltpu.sync_copy(idxs.at[pl.ds(wid * epw * FIELDS, epw * FIELDS)], idxbuf)

    def gather(c, half, sem):
        return pltpu.make_async_copy(
            tbl.at[idxbuf.at[pl.ds(c * CROWS, CROWS)]],
            ebuf.at[pl.ds(half * CROWS, CROWS)],
            sem,
        )

    gather(0, 0, sem0).start()

    def elem(el, carry, cc, par):
        outv = carry
        row0 = par * CROWS + el * FIELDS
        accs = [jnp.zeros((DIM,), jnp.float32) for _ in range(8)]
        for p, (i, j) in enumerate(PAIRS):
            a = ebuf[row0 + i, pl.ds(j * DIM, DIM)]
            b = ebuf[row0 + j, pl.ds(i * DIM, DIM)]
            accs[p % 8] = accs[p % 8] + a * b
        acc = ((accs[0] + accs[1]) + (accs[2] + accs[3])) + (
            (accs[4] + accs[5]) + (accs[6] + accs[7]))
        s = _lane_total(acc)
        eg = cc * G + el
        lane = lax.rem(eg, 16)
        outv = outv + jnp.where(lax.iota(jnp.int32, 16) == lane, s, 0.0)

        @pl.when(lane == 15)
        def _():
            outbuf[pl.ds(eg - 15, 16)] = outv

        return jnp.where(lane == 15, jnp.zeros_like(outv), outv)

    def chunk(cc, outv):
        par = lax.rem(cc, 2)
        nxt = cc + 1

        @pl.when(par == 0)
        def _():
            gather(cc, 0, sem0).wait()

            @pl.when(nxt < chunks)
            def _():
                gather(nxt, 1, sem1).start()

        @pl.when(par == 1)
        def _():
            gather(cc, 1, sem1).wait()

            @pl.when(nxt < chunks)
            def _():
                gather(nxt, 0, sem0).start()

        return lax.fori_loop(
            0, G, functools.partial(elem, cc=cc, par=par), outv)

    lax.fori_loop(0, chunks, chunk, jnp.zeros((16,), jnp.float32))
    pltpu.sync_copy(outbuf, out.at[pl.ds(wid * epw, epw)])


def kernel(inputs, W):
    batch, fields = inputs.shape
    feature = W.shape[0]
    tbl = W.reshape(feature, ROW)
    idx = inputs.reshape(-1)
    epw = batch // NW

    run = functools.partial(
        pl.kernel,
        out_type=jax.ShapeDtypeStruct((batch,), jnp.float32),
        mesh=plsc.VectorSubcoreMesh(core_axis_name="c", subcore_axis_name="s"),
        scratch_types=[
            pltpu.VMEM((epw * FIELDS,), jnp.int32),
            pltpu.VMEM((2 * CROWS, ROW), jnp.float32),
            pltpu.VMEM((epw,), jnp.float32),
            pltpu.SemaphoreType.DMA,
            pltpu.SemaphoreType.DMA,
        ],
        compiler_params=pltpu.CompilerParams(use_tc_tiling_on_sc=False),
    )(functools.partial(_ffm_body, batch))
    out = run(tbl, idx)
    return out.reshape(batch, 1)


# speedup vs baseline: 1.4646x; 1.0487x over previous
import functools

import jax
import jax.numpy as jnp
from jax import lax
from jax.experimental import pallas as pl
from jax.experimental.pallas import tpu as pltpu
from jax.experimental.pallas import tpu_sc as plsc

FIELDS = 26
DIM = 16
ROW = FIELDS * DIM
NC, NS = 2, 16
NW = NC * NS

G = 4
CROWS = G * FIELDS

PAIRS = [(i, j) for i in range(FIELDS - 1) for j in range(i + 1, FIELDS)]


def _lane_total(v):
    dnums = lax.GatherDimensionNumbers(
        offset_dims=(), collapsed_slice_dims=(0,), start_index_map=(0,))
    for sh in (8, 4, 2, 1):
        perm = jnp.bitwise_xor(lax.iota(jnp.int32, 16), sh)
        v = v + lax.gather(
            v, perm[:, None], dimension_numbers=dnums, slice_sizes=(1,),
            mode=lax.GatherScatterMode.PROMISE_IN_BOUNDS)
    return v


def _ffm_body(batch, tbl, idxs, out, idxbuf, ebuf, outbuf, sem0, sem1):
    epw = batch // NW
    chunks = epw // G
    wid = lax.axis_index("s") * NC + lax.axis_index("c")

    pltpu.sync_copy(idxs.at[pl.ds(wid * epw * FIELDS, epw * FIELDS)], idxbuf)

    def gather(c, half, sem):
        return pltpu.make_async_copy(
            tbl.at[idxbuf.at[pl.ds(c * CROWS, CROWS)]],
            ebuf.at[pl.ds(half * CROWS, CROWS)],
            sem,
        )

    gather(0, 0, sem0).start()

    def elem(el, carry, cc, par):
        outv = carry
        row0 = par * CROWS + el * FIELDS
        accs = [jnp.zeros((DIM,), jnp.float32) for _ in range(4)]
        for delta in range(1, FIELDS):
            n = FIELDS - delta
            d16 = delta * DIM

            def pair2(t, c, delta=delta):
                ra, ca, cb, a0, a1 = c
                x0 = ebuf[ra, pl.ds(ca, DIM)]
                y0 = ebuf[ra + delta, pl.ds(cb, DIM)]
                x1 = ebuf[ra + 1, pl.ds(ca + DIM, DIM)]
                y1 = ebuf[ra + 1 + delta, pl.ds(cb + DIM, DIM)]
                return (ra + 2, ca + 2 * DIM, cb + 2 * DIM,
                        a0 + x0 * y0, a1 + x1 * y1)

            _, _, _, a0, a1 = lax.fori_loop(
                0, n // 2, pair2, (row0, d16, 0, accs[0], accs[1]))
            accs[0], accs[1] = a0, a1
            if n % 2:
                k = n - 1
                x = ebuf[row0 + k, pl.ds((k + delta) * DIM, DIM)]
                y = ebuf[row0 + k + delta, pl.ds(k * DIM, DIM)]
                accs[2 + (delta % 2)] = accs[2 + (delta % 2)] + x * y
        acc = (accs[0] + accs[1]) + (accs[2] + accs[3])
        s = _lane_total(acc)
        eg = cc * G + el
        lane = lax.rem(eg, 16)
        outv = outv + jnp.where(lax.iota(jnp.int32, 16) == lane, s, 0.0)

        @pl.when(lane == 15)
        def _():
            outbuf[pl.ds(eg - 15, 16)] = outv

        return jnp.where(lane == 15, jnp.zeros_like(outv), outv)

    def chunk(cc, outv):
        par = lax.rem(cc, 2)
        nxt = cc + 1

        @pl.when(par == 0)
        def _():
            gather(cc, 0, sem0).wait()

            @pl.when(nxt < chunks)
            def _():
                gather(nxt, 1, sem1).start()

        @pl.when(par == 1)
        def _():
            gather(cc, 1, sem1).wait()

            @pl.when(nxt < chunks)
            def _():
                gather(nxt, 0, sem0).start()

        return lax.fori_loop(
            0, G, functools.partial(elem, cc=cc, par=par), outv)

    lax.fori_loop(0, chunks, chunk, jnp.zeros((16,), jnp.float32))
    pltpu.sync_copy(outbuf, out.at[pl.ds(wid * epw, epw)])


def kernel(inputs, W):
    batch, fields = inputs.shape
    feature = W.shape[0]
    tbl = W.reshape(feature, ROW)
    idx = inputs.reshape(-1)
    epw = batch // NW

    run = functools.partial(
        pl.kernel,
        out_type=jax.ShapeDtypeStruct((batch,), jnp.float32),
        mesh=plsc.VectorSubcoreMesh(core_axis_name="c", subcore_axis_name="s"),
        scratch_types=[
            pltpu.VMEM((epw * FIELDS,), jnp.int32),
            pltpu.VMEM((2 * CROWS, ROW), jnp.float32),
            pltpu.VMEM((epw,), jnp.float32),
            pltpu.SemaphoreType.DMA,
            pltpu.SemaphoreType.DMA,
        ],
        compiler_params=pltpu.CompilerParams(use_tc_tiling_on_sc=False),
    )(functools.partial(_ffm_body, batch))
    out = run(tbl, idx)
    return out.reshape(batch, 1)
